# baseline (speedup 1.0000x reference)
import functools

import jax
import jax.numpy as jnp
from jax import lax
from jax.experimental import pallas as pl
from jax.experimental.pallas import tpu as pltpu
from jax.experimental.pallas import tpu_sc as plsc

N_CLASS = 1000000
EMBED_DIM = 32
BATCH = 16384
FIELDS = 26

_NC, _NS = 2, 16
_NW = _NC * _NS
_NPAIR = FIELDS * EMBED_DIM
_PPW = _NPAIR // _NW

_mesh = plsc.VectorSubcoreMesh(core_axis_name="c", subcore_axis_name="s")


@functools.partial(
    pl.kernel,
    mesh=_mesh,
    compiler_params=pltpu.CompilerParams(use_tc_tiling_on_sc=False),
    out_type=jax.ShapeDtypeStruct((FIELDS, EMBED_DIM, BATCH), jnp.float32),
    scratch_types=[
        pltpu.VMEM((2, BATCH), jnp.int32),
        pltpu.VMEM((2, BATCH), jnp.float32),
        pltpu.SemaphoreType.DMA,
    ],
)
def _gather_kernel(idx_hbm, table_hbm, out_hbm, idx_v, rows_v, gsem):
    wid = lax.axis_index("s") * _NC + lax.axis_index("c")
    p_lo = wid * _PPW
    p_hi = p_lo + _PPW
    f0 = p_lo // EMBED_DIM
    f1 = (p_hi - 1) // EMBED_DIM
    pltpu.sync_copy(idx_hbm.at[f0], idx_v.at[0])
    pltpu.sync_copy(idx_hbm.at[f1], idx_v.at[1])

    d0 = lax.rem(p_lo, EMBED_DIM)
    pltpu.async_copy(table_hbm.at[d0].at[idx_v.at[0]], rows_v.at[0], gsem)

    def body(k, carry):
        p = p_lo + k
        buf = lax.rem(k, 2)
        f = p // EMBED_DIM
        d = lax.rem(p, EMBED_DIM)
        pltpu.make_async_copy(
            table_hbm.at[d].at[idx_v.at[f - f0]], rows_v.at[buf], gsem
        ).wait()

        @pl.when(k + 1 < _PPW)
        def _():
            p2 = p + 1
            f2 = p2 // EMBED_DIM
            d2 = lax.rem(p2, EMBED_DIM)
            pltpu.async_copy(
                table_hbm.at[d2].at[idx_v.at[f2 - f0]],
                rows_v.at[lax.rem(k + 1, 2)],
                gsem,
            )

        pltpu.sync_copy(rows_v.at[buf], out_hbm.at[f, d])
        return carry

    lax.fori_loop(0, _PPW, body, 0)


def kernel(z_category, categ_embed_weight):
    idx_t = z_category.T.astype(jnp.int32)
    table_t = categ_embed_weight.T
    out_t = _gather_kernel(idx_t, table_t)
    return out_t.transpose(2, 0, 1)


# speedup vs baseline: 2.3584x; 2.3584x over previous
import functools

import jax
import jax.numpy as jnp
from jax import lax
from jax.experimental import pallas as pl
from jax.experimental.pallas import tpu as pltpu
from jax.experimental.pallas import tpu_sc as plsc

N_CLASS = 1000000
EMBED_DIM = 32
BATCH = 16384
FIELDS = 26

_B = BATCH * FIELDS
_NC, _NS = 2, 16
_NW = _NC * _NS

_LANES = 16
_NBLK = N_CLASS // 128
_NBLK_REM = N_CLASS - _NBLK * 128
_BLK_PER_W = _NBLK // _NW
_BLK_EXTRA = _NBLK % _NW

_mesh = plsc.VectorSubcoreMesh(core_axis_name="c", subcore_axis_name="s")


@functools.partial(
    pl.kernel,
    mesh=_mesh,
    compiler_params=pltpu.CompilerParams(
        use_tc_tiling_on_sc=True, needs_layout_passes=False
    ),
    out_type=jax.ShapeDtypeStruct((N_CLASS // 4, 128), jnp.float32),
    scratch_types=[
        pltpu.VMEM((EMBED_DIM, 128), jnp.float32),
        pltpu.VMEM((EMBED_DIM, 128), jnp.float32),
    ],
)
def _transpose_kernel(table_t_hbm, tail_hbm, out_hbm, vblk, tblk):
    wid = lax.axis_index("s") * _NC + lax.axis_index("c")
    lo = wid * _BLK_PER_W + jnp.minimum(wid, _BLK_EXTRA)
    hi = lo + _BLK_PER_W + jnp.where(wid < _BLK_EXTRA, 1, 0)

    iota = lax.iota(jnp.int32, _LANES)

    def transpose_block(ncols):
        def prow(p, carry):
            for g in range(8):
                rows = lax.rem(16 * g, EMBED_DIM) + iota
                cols = jnp.full((_LANES,), 4 * p + (16 * g) // EMBED_DIM,
                                jnp.int32)
                vals = plsc.load_gather(vblk, [rows, cols])
                tblk[p, pl.ds(16 * g, 16)] = vals
            return carry
        lax.fori_loop(0, ncols // 4, prow, 0)

    def body(b, carry):
        pltpu.sync_copy(table_t_hbm.at[:, pl.ds(b * 128, 128)], vblk)
        transpose_block(128)
        pltpu.sync_copy(tblk, out_hbm.at[pl.ds(b * 32, 32)])
        return carry

    lax.fori_loop(lo, hi, body, 0)

    @pl.when(wid == 0)
    def _():
        pltpu.sync_copy(tail_hbm, tblk.at[pl.ds(0, _NBLK_REM // 4)])
        pltpu.sync_copy(
            tblk.at[pl.ds(0, _NBLK_REM // 4)],
            out_hbm.at[pl.ds(_NBLK * 32, _NBLK_REM // 4)],
        )


_BPW = _B // _NW
_CHUNK = 512
_NCHUNK = _BPW // _CHUNK


@functools.partial(
    pl.kernel,
    mesh=_mesh,
    compiler_params=pltpu.CompilerParams(use_tc_tiling_on_sc=False),
    out_type=jax.ShapeDtypeStruct((_B, EMBED_DIM), jnp.float32),
    scratch_types=[
        pltpu.VMEM((_BPW,), jnp.int32),
        pltpu.VMEM((_CHUNK, EMBED_DIM), jnp.float32),
        pltpu.SemaphoreType.DMA,
    ],
)
def _gather_kernel(idx_hbm, table_hbm, out_hbm, idx_v, rows_v, sem):
    wid = lax.axis_index("s") * _NC + lax.axis_index("c")
    base = wid * _BPW
    pltpu.sync_copy(idx_hbm.at[pl.ds(base, _BPW)], idx_v)

    def body(c, carry):
        off = c * _CHUNK
        pltpu.async_copy(
            table_hbm.at[idx_v.at[pl.ds(off, _CHUNK)]], rows_v, sem
        ).wait()
        pltpu.sync_copy(rows_v, out_hbm.at[pl.ds(base + off, _CHUNK)])
        return carry

    lax.fori_loop(0, _NCHUNK, body, 0)


def kernel(z_category, categ_embed_weight):
    idx = z_category.reshape(-1).astype(jnp.int32)
    tail = categ_embed_weight[_NBLK * 128:].reshape(_NBLK_REM // 4, 128)
    table_packed = _transpose_kernel(categ_embed_weight.T, tail)
    table_rm = table_packed.reshape(N_CLASS, EMBED_DIM)
    out = _gather_kernel(idx, table_rm)
    return out.reshape(z_category.shape + (EMBED_DIM,))


# speedup vs baseline: 3.3301x; 1.4121x over previous
import functools

import jax
import jax.numpy as jnp
from jax import lax
from jax.experimental import pallas as pl
from jax.experimental.pallas import tpu as pltpu
from jax.experimental.pallas import tpu_sc as plsc

N_CLASS = 1000000
EMBED_DIM = 32
BATCH = 16384
FIELDS = 26

_B = BATCH * FIELDS
_NC, _NS = 2, 16
_NW = _NC * _NS

_LANES = 16
_TW = 512
_NBLK = N_CLASS // _TW
_NBLK_REM = N_CLASS - _NBLK * _TW
_BLK_PER_W = _NBLK // _NW
_BLK_EXTRA = _NBLK % _NW

_mesh = plsc.VectorSubcoreMesh(core_axis_name="c", subcore_axis_name="s")


@functools.partial(
    pl.kernel,
    mesh=_mesh,
    compiler_params=pltpu.CompilerParams(
        use_tc_tiling_on_sc=True, needs_layout_passes=False
    ),
    out_type=jax.ShapeDtypeStruct((N_CLASS // 4, 128), jnp.float32),
    scratch_types=[
        pltpu.VMEM((2, EMBED_DIM, _TW), jnp.float32),
        pltpu.VMEM((2, _TW // 4, 128), jnp.float32),
        pltpu.SemaphoreType.DMA,
        pltpu.SemaphoreType.DMA,
    ],
)
def _transpose_kernel(table_t_hbm, tail_hbm, out_hbm, vblk, tblk, isem, osem):
    wid = lax.axis_index("s") * _NC + lax.axis_index("c")
    lo = wid * _BLK_PER_W + jnp.minimum(wid, _BLK_EXTRA)
    hi = lo + _BLK_PER_W + jnp.where(wid < _BLK_EXTRA, 1, 0)

    iota = lax.iota(jnp.int32, _LANES)
    colsbase = lax.rem(iota, 4) * EMBED_DIM
    rows_g = [16 * g // 4 + iota // 4 for g in range(_TW // _LANES)]

    def issue_in(b, buf):
        return pltpu.async_copy(
            table_t_hbm.at[:, pl.ds(b * _TW, _TW)], vblk.at[buf], isem
        )

    def issue_out(b, buf):
        return pltpu.async_copy(
            tblk.at[buf], out_hbm.at[pl.ds(b * (_TW // 4), _TW // 4)], osem
        )

    def compute(buf):
        def drow(d, carry):
            cols_d = colsbase + d
            for g in range(_TW // _LANES):
                vals = vblk[buf, d, pl.ds(16 * g, 16)]
                plsc.store_scatter(tblk.at[buf], [rows_g[g], cols_d], vals)
            return carry
        lax.fori_loop(0, EMBED_DIM, drow, 0)

    issue_in(lo, 0)

    def body(k, carry):
        buf = lax.rem(k - lo, 2)
        pltpu.make_async_copy(
            table_t_hbm.at[:, pl.ds(k * _TW, _TW)], vblk.at[buf], isem
        ).wait()

        @pl.when(k + 1 < hi)
        def _():
            issue_in(k + 1, lax.rem(k + 1 - lo, 2))

        @pl.when(k - 2 >= lo)
        def _():
            pltpu.make_async_copy(
                tblk.at[buf],
                out_hbm.at[pl.ds((k - 2) * (_TW // 4), _TW // 4)],
                osem,
            ).wait()

        compute(buf)
        issue_out(k, buf)
        return carry

    lax.fori_loop(lo, hi, body, 0)

    def drain(k, carry):
        pltpu.make_async_copy(
            tblk.at[lax.rem(k - lo, 2)],
            out_hbm.at[pl.ds(k * (_TW // 4), _TW // 4)],
            osem,
        ).wait()
        return carry

    lax.fori_loop(jnp.maximum(lo, hi - 2), hi, drain, 0)

    @pl.when(wid == 0)
    def _():
        pltpu.sync_copy(tail_hbm, tblk.at[0, pl.ds(0, _NBLK_REM // 4)])
        pltpu.sync_copy(
            tblk.at[0, pl.ds(0, _NBLK_REM // 4)],
            out_hbm.at[pl.ds(_NBLK * (_TW // 4), _NBLK_REM // 4)],
        )


_BPW = _B // _NW
_CHUNK = 512
_NCHUNK = _BPW // _CHUNK


@functools.partial(
    pl.kernel,
    mesh=_mesh,
    compiler_params=pltpu.CompilerParams(use_tc_tiling_on_sc=False),
    out_type=jax.ShapeDtypeStruct((_B, EMBED_DIM), jnp.float32),
    scratch_types=[
        pltpu.VMEM((_BPW,), jnp.int32),
        pltpu.VMEM((_CHUNK, EMBED_DIM), jnp.float32),
        pltpu.SemaphoreType.DMA,
    ],
)
def _gather_kernel(idx_hbm, table_hbm, out_hbm, idx_v, rows_v, sem):
    wid = lax.axis_index("s") * _NC + lax.axis_index("c")
    base = wid * _BPW
    pltpu.sync_copy(idx_hbm.at[pl.ds(base, _BPW)], idx_v)

    def body(c, carry):
        off = c * _CHUNK
        pltpu.async_copy(
            table_hbm.at[idx_v.at[pl.ds(off, _CHUNK)]], rows_v, sem
        ).wait()
        pltpu.sync_copy(rows_v, out_hbm.at[pl.ds(base + off, _CHUNK)])
        return carry

    lax.fori_loop(0, _NCHUNK, body, 0)


def kernel(z_category, categ_embed_weight):
    idx = z_category.reshape(-1).astype(jnp.int32)
    tail = categ_embed_weight[_NBLK * _TW:].reshape(_NBLK_REM // 4, 128)
    table_packed = _transpose_kernel(categ_embed_weight.T, tail)
    table_rm = table_packed.reshape(N_CLASS, EMBED_DIM)
    out = _gather_kernel(idx, table_rm)
    return out.reshape(z_category.shape + (EMBED_DIM,))
